# baseline (speedup 1.0000x reference)
import functools
import jax
import jax.numpy as jnp
from jax import lax
from jax.experimental import pallas as pl
from jax.experimental.pallas import tpu as pltpu
from jax.experimental.pallas import tpu_sc as plsc

_DIM = 128
_BASE = 10000.0

_ROW = 2 * 16 * _DIM
_NW = 32
_P = 4
_RING = 3
_CH = _P * _ROW
_CS_ROW = 2 * _DIM
_CS_CH = _P * _CS_ROW


@functools.lru_cache(maxsize=None)
def _cs_table(sq):
    freqs = 1.0 / (_BASE ** (jnp.arange(0, _DIM, 2)[: _DIM // 2].astype(jnp.float32) / _DIM))
    t = jnp.arange(sq).astype(jnp.float32)
    f = jnp.outer(t, freqs)
    cos = jnp.cos(f)
    sin = jnp.sin(f)
    c_full = jnp.repeat(cos, 2, axis=1)
    s_full = jnp.stack([-sin, sin], axis=-1).reshape(sq, _DIM)
    return jnp.concatenate([c_full, s_full], axis=1).reshape(-1)


def _make_sc_rope(sq):
    pos_per_w = sq // _NW
    chunks = pos_per_w // _P
    n = sq * _ROW
    mesh = plsc.VectorSubcoreMesh(core_axis_name="c", subcore_axis_name="s")

    @functools.partial(
        pl.kernel,
        mesh=mesh,
        out_type=[
            jax.ShapeDtypeStruct((n,), jnp.float32),
            jax.ShapeDtypeStruct((n,), jnp.float32),
        ],
        scratch_types=[pltpu.VMEM((_CH,), jnp.float32)] * _RING
        + [pltpu.VMEM((_CH,), jnp.float32)] * _RING
        + [pltpu.VMEM((_CS_CH,), jnp.float32)] * _RING
        + [pltpu.SemaphoreType.DMA] * (2 * _RING),
    )
    def rope_sc(q_hbm, k_hbm, cs_hbm, qo_hbm, ko_hbm, *scratch):
        qbufs = scratch[:_RING]
        kbufs = scratch[_RING:2 * _RING]
        csbufs = scratch[2 * _RING:3 * _RING]
        in_sems = scratch[3 * _RING:4 * _RING]
        out_sems = scratch[4 * _RING:5 * _RING]
        wid = lax.axis_index("s") * 2 + lax.axis_index("c")
        start_pos = wid * pos_per_w
        swp = jnp.bitwise_xor(lax.iota(jnp.int32, 16), 1)
        swp_idx = swp.reshape(16, 1)
        gather_dnums = lax.GatherDimensionNumbers(
            offset_dims=(), collapsed_slice_dims=(0,), start_index_map=(0,))

        def pair_swap(x):
            return lax.gather(
                x, swp_idx, gather_dnums, (1,),
                mode=lax.GatherScatterMode.PROMISE_IN_BOUNDS)

        def in_copies(g, b):
            base = (start_pos + g * _P) * _ROW
            cbase = (start_pos + g * _P) * _CS_ROW
            return (
                pltpu.make_async_copy(q_hbm.at[pl.ds(base, _CH)], qbufs[b], in_sems[b]),
                pltpu.make_async_copy(k_hbm.at[pl.ds(base, _CH)], kbufs[b], in_sems[b]),
                pltpu.make_async_copy(cs_hbm.at[pl.ds(cbase, _CS_CH)], csbufs[b], in_sems[b]),
            )

        def out_copies(g, b):
            base = (start_pos + g * _P) * _ROW
            return (
                pltpu.make_async_copy(qbufs[b], qo_hbm.at[pl.ds(base, _CH)], out_sems[b]),
                pltpu.make_async_copy(kbufs[b], ko_hbm.at[pl.ds(base, _CH)], out_sems[b]),
            )

        def start_in(g, b):
            for c in in_copies(g, b):
                c.start()

        def wait_in(g, b):
            for c in in_copies(g, b):
                c.wait()

        def start_out(g, b):
            for c in out_copies(g, b):
                c.start()

        def wait_out(g, b):
            for c in out_copies(g, b):
                c.wait()

        def compute(b):
            qb = qbufs[b]
            kb = kbufs[b]
            csb = csbufs[b]
            for pos in range(_P):
                for t in range(_DIM // 16):
                    cvec = csb[pl.ds(pos * _CS_ROW + t * 16, 16)]
                    svec = csb[pl.ds(pos * _CS_ROW + _DIM + t * 16, 16)]

                    def hbody(h, carry, pos=pos, t=t, cvec=cvec, svec=svec):
                        addr = pos * _ROW + h * _DIM + t * 16
                        for buf in (qb, kb):
                            x = buf[pl.ds(addr, 16)]
                            sw = pair_swap(x)
                            buf[pl.ds(addr, 16)] = x * cvec + sw * svec
                        return carry

                    lax.fori_loop(0, _ROW // _DIM, hbody, 0, unroll=2)

        start_in(0, 0)
        start_in(1, 1)

        def outer(i, carry):
            gbase = i * _RING
            for b in range(_RING):
                g = gbase + b

                @pl.when(g < chunks)
                def _():
                    bp = (b - 1) % _RING
                    wait_in(g, b)
                    compute(b)
                    start_out(g, b)

                    @pl.when(g >= 1)
                    def _():
                        wait_out(g - 1, bp)

                    @pl.when(g + 2 < chunks)
                    def _():
                        start_in(g + 2, bp)

            return carry

        n_outer = (chunks + _RING - 1) // _RING
        lax.fori_loop(0, n_outer, outer, 0)
        wait_out(chunks - 1, (chunks - 1) % _RING)

    return rope_sc


def kernel(query, key):
    sq, bsz, nh, hh = query.shape
    n = sq * bsz * nh * hh
    cs = _cs_table(sq)
    qo, ko = _make_sc_rope(sq)(query.reshape(n), key.reshape(n), cs)
    return qo.reshape(query.shape), ko.reshape(key.shape)


# speedup vs baseline: 3.6360x; 3.6360x over previous
import functools
import jax
import jax.numpy as jnp
from jax import lax
from jax.experimental import pallas as pl
from jax.experimental.pallas import tpu as pltpu
from jax.experimental.pallas import tpu_sc as plsc

_DIM = 128
_BASE = 10000.0

_ROW = 2 * 16 * _DIM
_NW = 32
_P = 4
_RING = 3
_CH = _P * _ROW
_CS_ROW = 2 * _DIM
_CS_CH = _P * _CS_ROW


@functools.lru_cache(maxsize=None)
def _cs_table(sq):
    freqs = 1.0 / (_BASE ** (jnp.arange(0, _DIM, 2)[: _DIM // 2].astype(jnp.float32) / _DIM))
    t = jnp.arange(sq).astype(jnp.float32)
    f = jnp.outer(t, freqs)
    cos = jnp.cos(f)
    sin = jnp.sin(f)
    c_full = jnp.repeat(cos, 2, axis=1)
    s_full = jnp.stack([-sin, sin], axis=-1).reshape(sq, _DIM)
    return jnp.concatenate([c_full, s_full], axis=1).reshape(-1)


def _make_sc_rope(sq):
    pos_per_w = sq // _NW
    chunks = pos_per_w // _P
    n = sq * _ROW
    mesh = plsc.VectorSubcoreMesh(core_axis_name="c", subcore_axis_name="s")

    @functools.partial(
        pl.kernel,
        mesh=mesh,
        out_type=[
            jax.ShapeDtypeStruct((n,), jnp.float32),
            jax.ShapeDtypeStruct((n,), jnp.float32),
        ],
        scratch_types=[pltpu.VMEM((_CH,), jnp.float32)] * _RING
        + [pltpu.VMEM((_CH,), jnp.float32)] * _RING
        + [pltpu.VMEM((_CS_CH,), jnp.float32)] * _RING
        + [pltpu.SemaphoreType.DMA] * (2 * _RING),
    )
    def rope_sc(q_hbm, k_hbm, cs_hbm, qo_hbm, ko_hbm, *scratch):
        qbufs = scratch[:_RING]
        kbufs = scratch[_RING:2 * _RING]
        csbufs = scratch[2 * _RING:3 * _RING]
        in_sems = scratch[3 * _RING:4 * _RING]
        out_sems = scratch[4 * _RING:5 * _RING]
        wid = lax.axis_index("s") * 2 + lax.axis_index("c")
        start_pos = wid * pos_per_w
        swp = jnp.bitwise_xor(lax.iota(jnp.int32, 16), 1)
        swp_idx = swp.reshape(16, 1)
        gather_dnums = lax.GatherDimensionNumbers(
            offset_dims=(), collapsed_slice_dims=(0,), start_index_map=(0,))

        def pair_swap(x):
            return lax.gather(
                x, swp_idx, gather_dnums, (1,),
                mode=lax.GatherScatterMode.PROMISE_IN_BOUNDS)

        def in_copies(g, b):
            base = (start_pos + g * _P) * _ROW
            cbase = (start_pos + g * _P) * _CS_ROW
            return (
                pltpu.make_async_copy(q_hbm.at[pl.ds(base, _CH)], qbufs[b], in_sems[b]),
                pltpu.make_async_copy(k_hbm.at[pl.ds(base, _CH)], kbufs[b], in_sems[b]),
                pltpu.make_async_copy(cs_hbm.at[pl.ds(cbase, _CS_CH)], csbufs[b], in_sems[b]),
            )

        def out_copies(g, b):
            base = (start_pos + g * _P) * _ROW
            return (
                pltpu.make_async_copy(qbufs[b], qo_hbm.at[pl.ds(base, _CH)], out_sems[b]),
                pltpu.make_async_copy(kbufs[b], ko_hbm.at[pl.ds(base, _CH)], out_sems[b]),
            )

        def start_in(g, b):
            for c in in_copies(g, b):
                c.start()

        def wait_in(g, b):
            for c in in_copies(g, b):
                c.wait()

        def start_out(g, b):
            for c in out_copies(g, b):
                c.start()

        def wait_out(g, b):
            for c in out_copies(g, b):
                c.wait()

        def compute(b):
            qb = qbufs[b]
            kb = kbufs[b]
            csb = csbufs[b]

            @plsc.parallel_loop(0, _P)
            def _pos_loop(pos):
                for t in range(_DIM // 16):
                    cvec = csb[pl.ds(pos * _CS_ROW + t * 16, 16)]
                    svec = csb[pl.ds(pos * _CS_ROW + _DIM + t * 16, 16)]

                    @plsc.parallel_loop(0, _ROW // _DIM, unroll=4)
                    def _h_loop(h, pos=pos, t=t, cvec=cvec, svec=svec):
                        addr = pos * _ROW + h * _DIM + t * 16
                        for buf in (qb, kb):
                            x = buf[pl.ds(addr, 16)]
                            sw = pair_swap(x)
                            buf[pl.ds(addr, 16)] = x * cvec + sw * svec

        start_in(0, 0)
        start_in(1, 1)

        def outer(i, carry):
            gbase = i * _RING
            for b in range(_RING):
                g = gbase + b

                @pl.when(g < chunks)
                def _():
                    bp = (b - 1) % _RING
                    wait_in(g, b)
                    compute(b)
                    start_out(g, b)

                    @pl.when(g >= 1)
                    def _():
                        wait_out(g - 1, bp)

                    @pl.when(g + 2 < chunks)
                    def _():
                        start_in(g + 2, bp)

            return carry

        n_outer = (chunks + _RING - 1) // _RING
        lax.fori_loop(0, n_outer, outer, 0)
        wait_out(chunks - 1, (chunks - 1) % _RING)

    return rope_sc


def kernel(query, key):
    sq, bsz, nh, hh = query.shape
    n = sq * bsz * nh * hh
    cs = _cs_table(sq)
    qo, ko = _make_sc_rope(sq)(query.reshape(n), key.reshape(n), cs)
    return qo.reshape(query.shape), ko.reshape(key.shape)
